# baseline (speedup 1.0000x reference)
import functools

import jax
import jax.numpy as jnp
from jax import lax
from jax.experimental import pallas as pl
from jax.experimental.pallas import tpu as pltpu
from jax.experimental.pallas import tpu_sc as plsc

BATCH = 16384
EMBEDDING_DIM = 64

_info = plsc.get_sparse_core_info()
_NC, _NS = _info.num_cores, _info.num_subcores
_NW = _NC * _NS
_B_PER_W = BATCH // _NW


@functools.partial(
    pl.kernel,
    mesh=plsc.VectorSubcoreMesh(core_axis_name="c", subcore_axis_name="s"),
    out_type=jax.ShapeDtypeStruct((BATCH, EMBEDDING_DIM), jnp.float32),
    scratch_types=[
        pltpu.VMEM((_B_PER_W,), jnp.int32),
        pltpu.VMEM((_B_PER_W, EMBEDDING_DIM), jnp.float32),
        pltpu.SemaphoreType.DMA,
    ],
    compiler_params=pltpu.CompilerParams(use_tc_tiling_on_sc=False),
)
def _gather_kernel(idx_hbm, table_hbm, out_hbm, idx_v, rows_v, sem):
    wid = lax.axis_index("s") * _NC + lax.axis_index("c")
    base = wid * _B_PER_W
    pltpu.sync_copy(idx_hbm.at[pl.ds(base, _B_PER_W)], idx_v)
    pltpu.async_copy(table_hbm.at[idx_v], rows_v, sem).wait()
    pltpu.sync_copy(rows_v, out_hbm.at[pl.ds(base, _B_PER_W)])


def kernel(x, embedding_location):
    location_idx = x[:, 0]
    return _gather_kernel(location_idx, embedding_location)


# speedup vs baseline: 3.0201x; 3.0201x over previous
import functools

import jax
import jax.numpy as jnp
from jax import lax
from jax.experimental import pallas as pl
from jax.experimental.pallas import tpu as pltpu
from jax.experimental.pallas import tpu_sc as plsc

BATCH = 16384
EMBEDDING_DIM = 64
NUM_LOCATION = 1000000

_info = plsc.get_sparse_core_info()
_NC, _NS = _info.num_cores, _info.num_subcores
_L = 16
_B_PER_S = BATCH // _NS
_CH = 27776
_NMAIN = 36
_TAIL0 = _NMAIN * _CH
_NTAIL = NUM_LOCATION - _TAIL0
_NCHUNK = _NMAIN + 1
_NCPAD = 48
_CPS = EMBEDDING_DIM // _NC // _NS


@functools.partial(
    pl.kernel,
    mesh=plsc.VectorSubcoreMesh(core_axis_name="c", subcore_axis_name="s"),
    out_type=jax.ShapeDtypeStruct((8, 8, BATCH), jnp.float32),
    scratch_types=[
        pltpu.VMEM((_B_PER_S,), jnp.int32),
        pltpu.VMEM((_B_PER_S + _L,), jnp.int32),
        pltpu.VMEM((_NCHUNK * _L,), jnp.int32),
        pltpu.VMEM((_NCHUNK * _L,), jnp.int32),
        pltpu.VMEM((_NCPAD,), jnp.int32),
        pltpu.VMEM((_NCPAD,), jnp.int32),
        pltpu.VMEM((_NS * _B_PER_S,), jnp.int32),
        pltpu.VMEM((_NS * _NCPAD,), jnp.int32),
        pltpu.VMEM((_NS * _NCPAD,), jnp.int32),
        pltpu.VMEM((_CH,), jnp.float32),
        pltpu.VMEM((_CH,), jnp.float32),
        pltpu.VMEM((BATCH,), jnp.float32),
        pltpu.VMEM((_NTAIL, EMBEDDING_DIM), jnp.float32),
        pltpu.VMEM_SHARED((_NS * _B_PER_S,), jnp.int32),
        pltpu.VMEM_SHARED((_NS * _NCPAD,), jnp.int32),
        pltpu.VMEM_SHARED((_NS * _NCPAD,), jnp.int32),
        pltpu.SemaphoreType.DMA,
        pltpu.SemaphoreType.DMA,
    ],
    compiler_params=pltpu.CompilerParams(
        use_tc_tiling_on_sc=True, needs_layout_passes=False
    ),
)
def _lookup_kernel(
    idx_hbm, t3_hbm, tail_hbm, out_hbm,
    idx_own, mybuf, mycnt, mystart, cnt_pk, start_pk,
    lists_all, cnt_all, start_all, buf0, buf1, vals_v, tail_v,
    lists_sh, cnt_sh, start_sh, sem0, sem1,
):
    core = lax.axis_index("c")
    sid = lax.axis_index("s")
    iota = lax.broadcasted_iota(jnp.int32, (_L,), 0)

    pltpu.sync_copy(idx_hbm.at[pl.ds(sid * _B_PER_S, _B_PER_S)], idx_own)
    pltpu.sync_copy(tail_hbm, tail_v)
    b_base = sid * _B_PER_S

    def _bucket(k, pos):
        lo = jnp.broadcast_to(k * _CH, (_L,)).astype(jnp.int32)
        hi = jnp.minimum(lo + _CH, NUM_LOCATION)
        pos_start = pos

        def _step(i, p):
            idx16 = idx_own[pl.ds(i * _L, _L)]
            m = (idx16 >= lo) & (idx16 < hi)
            ent16 = lax.shift_left(idx16 - lo, 16) | (b_base + i * _L + iota)
            ranks = plsc.cumsum(m.astype(jnp.int32))
            plsc.store_scatter(mybuf, [p + ranks - 1], ent16, mask=m)
            return p + jnp.max(ranks)

        pos = lax.fori_loop(0, _B_PER_S // _L, _step, pos)
        mycnt[pl.ds(k * _L, _L)] = jnp.broadcast_to(pos - pos_start, (_L,))
        mystart[pl.ds(k * _L, _L)] = jnp.broadcast_to(pos_start, (_L,))
        return pos

    lax.fori_loop(0, _NCHUNK, _bucket, jnp.int32(0))
    for t in range(_NCPAD // _L):
        sel = jnp.minimum((t * _L + iota) * _L, (_NCHUNK - 1) * _L)
        cnt_pk[pl.ds(t * _L, _L)] = plsc.load_gather(mycnt, [sel])
        start_pk[pl.ds(t * _L, _L)] = plsc.load_gather(mystart, [sel])

    pltpu.sync_copy(mybuf.at[pl.ds(0, _B_PER_S)],
                    lists_sh.at[pl.ds(sid * _B_PER_S, _B_PER_S)])
    pltpu.sync_copy(cnt_pk, cnt_sh.at[pl.ds(sid * _NCPAD, _NCPAD)])
    pltpu.sync_copy(start_pk, start_sh.at[pl.ds(sid * _NCPAD, _NCPAD)])
    plsc.subcore_barrier()
    pltpu.sync_copy(lists_sh, lists_all)
    pltpu.sync_copy(cnt_sh, cnt_all)
    pltpu.sync_copy(start_sh, start_all)

    bufs = (buf0, buf1)
    sems = (sem0, sem1)

    def _extract_bucket(k, src_load, sw):
        base_ck = sw * _NCPAD + k
        cnt16 = plsc.load_gather(cnt_all, [jnp.broadcast_to(base_ck, (_L,))])
        st16 = plsc.load_gather(start_all, [jnp.broadcast_to(base_ck, (_L,))])
        lbase = sw * _B_PER_S
        cnt = jnp.max(cnt16)
        st = jnp.max(st16)

        def _ext(jv, _2):
            lane = jv * _L + iota
            ent16 = plsc.load_gather(lists_all, [lbase + st + lane])
            lmask = lane < cnt16
            off16 = lax.shift_right_logical(ent16, 16)
            b16 = ent16 & 0xFFFF
            v = src_load(off16, lmask)
            plsc.store_scatter(vals_v, [b16], v, mask=lmask)
            return _2

        lax.fori_loop(0, (cnt + _L - 1) // _L, _ext, 0)

    for j in range(_CPS):
        comp = core * (_NS * _CPS) + sid * _CPS + j
        g = comp // 8
        s = comp % 8
        col = t3_hbm.at[g, s]
        pltpu.async_copy(col.at[pl.ds(0, _CH)], buf0, sem0)

        def _chunk_pair(k2, _, col=col):
            for slot in range(2):
                k = k2 * 2 + slot

                @pl.when(k + 1 < _NMAIN)
                def _prefetch(k=k, slot=slot, col=col):
                    pltpu.async_copy(
                        col.at[pl.ds((k + 1) * _CH, _CH)],
                        bufs[1 - slot], sems[1 - slot],
                    )

                pltpu.make_async_copy(
                    col.at[pl.ds(k * _CH, _CH)], bufs[slot], sems[slot]
                ).wait()
                for sw in range(_NS):
                    _extract_bucket(
                        k,
                        lambda o16, m, slot=slot: plsc.load_gather(
                            bufs[slot], [o16], mask=m),
                        sw,
                    )
            return _

        lax.fori_loop(0, _NMAIN // 2, _chunk_pair, 0)
        comp_v = jnp.broadcast_to(comp, (_L,)).astype(jnp.int32)
        for sw in range(_NS):
            _extract_bucket(
                _NMAIN,
                lambda o16, m: plsc.load_gather(
                    tail_v, [o16, comp_v], mask=m),
                sw,
            )
        pltpu.sync_copy(vals_v, out_hbm.at[g, s])


def kernel(x, embedding_location):
    location_idx = x[:, 0]
    t3 = embedding_location.T.reshape(8, 8, NUM_LOCATION)
    tail = embedding_location[_TAIL0:, :]
    out_t = _lookup_kernel(location_idx, t3, tail)
    return out_t.reshape(EMBEDDING_DIM, BATCH).T


# speedup vs baseline: 3.3456x; 1.1078x over previous
import functools

import jax
import jax.numpy as jnp
from jax import lax
from jax.experimental import pallas as pl
from jax.experimental.pallas import tpu as pltpu
from jax.experimental.pallas import tpu_sc as plsc

BATCH = 16384
EMBEDDING_DIM = 64
NUM_LOCATION = 1000000

_info = plsc.get_sparse_core_info()
_NC, _NS = _info.num_cores, _info.num_subcores
_L = 16
_B_PER_S = BATCH // _NS
_CH = 27776
_NMAIN = 36
_TAIL0 = _NMAIN * _CH
_NTAIL = NUM_LOCATION - _TAIL0
_NCHUNK = _NMAIN + 1
_NCPAD = 48
_SUPW = 5
_NSUP = 8
_CH1 = _SUPW * _CH
_CPS = EMBEDDING_DIM // _NC // _NS


@functools.partial(
    pl.kernel,
    mesh=plsc.VectorSubcoreMesh(core_axis_name="c", subcore_axis_name="s"),
    out_type=jax.ShapeDtypeStruct((8, 8, BATCH), jnp.float32),
    scratch_types=[
        pltpu.VMEM((_B_PER_S,), jnp.int32),
        pltpu.VMEM((_B_PER_S + _L,), jnp.int32),
        pltpu.VMEM((_B_PER_S + _L,), jnp.int32),
        pltpu.VMEM((_NSUP * _L,), jnp.int32),
        pltpu.VMEM((_NSUP * _L,), jnp.int32),
        pltpu.VMEM((_NCHUNK * _L,), jnp.int32),
        pltpu.VMEM((_NCHUNK * _L,), jnp.int32),
        pltpu.VMEM((_NCPAD,), jnp.int32),
        pltpu.VMEM((_NCPAD,), jnp.int32),
        pltpu.VMEM((_NS * _B_PER_S,), jnp.int32),
        pltpu.VMEM((_NS * _NCPAD,), jnp.int32),
        pltpu.VMEM((_NS * _NCPAD,), jnp.int32),
        pltpu.VMEM((_CH,), jnp.float32),
        pltpu.VMEM((_CH,), jnp.float32),
        pltpu.VMEM((BATCH,), jnp.float32),
        pltpu.VMEM((_NTAIL, EMBEDDING_DIM), jnp.float32),
        pltpu.VMEM_SHARED((_NS * _B_PER_S,), jnp.int32),
        pltpu.VMEM_SHARED((_NS * _NCPAD,), jnp.int32),
        pltpu.VMEM_SHARED((_NS * _NCPAD,), jnp.int32),
        pltpu.SemaphoreType.DMA,
        pltpu.SemaphoreType.DMA,
    ],
    compiler_params=pltpu.CompilerParams(
        use_tc_tiling_on_sc=True, needs_layout_passes=False
    ),
)
def _lookup_kernel(
    idx_hbm, t3_hbm, tail_hbm, out_hbm,
    idx_own, mybuf, supbuf, supcnt, supstart, mycnt, mystart, cnt_pk, start_pk,
    lists_all, cnt_all, start_all, buf0, buf1, vals_v, tail_v,
    lists_sh, cnt_sh, start_sh, sem0, sem1,
):
    core = lax.axis_index("c")
    sid = lax.axis_index("s")
    iota = lax.broadcasted_iota(jnp.int32, (_L,), 0)

    pltpu.sync_copy(idx_hbm.at[pl.ds(sid * _B_PER_S, _B_PER_S)], idx_own)
    pltpu.sync_copy(tail_hbm, tail_v)
    b_base = sid * _B_PER_S

    def _super(k1, pos):
        lo = jnp.broadcast_to(k1 * _CH1, (_L,)).astype(jnp.int32)
        hi = jnp.minimum(lo + _CH1, NUM_LOCATION)
        pos_start = pos

        def _step(i, p):
            idx16 = idx_own[pl.ds(i * _L, _L)]
            m = (idx16 >= lo) & (idx16 < hi)
            ent16 = lax.shift_left(idx16 - lo, 14) | (b_base + i * _L + iota)
            ranks = plsc.cumsum(m.astype(jnp.int32))
            plsc.store_scatter(supbuf, [p + ranks - 1], ent16, mask=m)
            return p + jnp.max(ranks)

        pos = lax.fori_loop(0, _B_PER_S // _L, _step, pos)
        supcnt[pl.ds(k1 * _L, _L)] = jnp.broadcast_to(pos - pos_start, (_L,))
        supstart[pl.ds(k1 * _L, _L)] = jnp.broadcast_to(pos_start, (_L,))
        return pos

    lax.fori_loop(0, _NSUP, _super, jnp.int32(0))

    def _split_super(k1, pos):
        scnt16 = supcnt[pl.ds(k1 * _L, _L)]
        sst16 = supstart[pl.ds(k1 * _L, _L)]
        scnt = jnp.max(scnt16)
        sst = jnp.max(sst16)
        nvec = (scnt + _L - 1) // _L

        def _sub(j2, pos):
            k = k1 * _SUPW + j2
            lo = jnp.broadcast_to(j2 * _CH, (_L,)).astype(jnp.int32)
            hi = jnp.minimum(lo + _CH, NUM_LOCATION - k1 * _CH1)
            pos_start = pos

            def _step2(i, p):
                lane = i * _L + iota
                ent = plsc.load_gather(supbuf, [sst + lane])
                loc = lax.shift_right_logical(ent, 14)
                m = (lane < scnt16) & (loc >= lo) & (loc < hi)
                ent2 = lax.shift_left(loc - lo, 16) | (ent & 0x3FFF)
                ranks = plsc.cumsum(m.astype(jnp.int32))
                plsc.store_scatter(mybuf, [p + ranks - 1], ent2, mask=m)
                return p + jnp.max(ranks)

            pos = lax.fori_loop(0, nvec, _step2, pos)
            mycnt[pl.ds(k * _L, _L)] = jnp.broadcast_to(pos - pos_start, (_L,))
            mystart[pl.ds(k * _L, _L)] = jnp.broadcast_to(pos_start, (_L,))
            return pos

        return lax.fori_loop(
            0, jnp.minimum(_SUPW, _NCHUNK - k1 * _SUPW), _sub, pos)

    lax.fori_loop(0, _NSUP, _split_super, jnp.int32(0))
    for t in range(_NCPAD // _L):
        sel = jnp.minimum((t * _L + iota) * _L, (_NCHUNK - 1) * _L)
        cnt_pk[pl.ds(t * _L, _L)] = plsc.load_gather(mycnt, [sel])
        start_pk[pl.ds(t * _L, _L)] = plsc.load_gather(mystart, [sel])

    pltpu.sync_copy(mybuf.at[pl.ds(0, _B_PER_S)],
                    lists_sh.at[pl.ds(sid * _B_PER_S, _B_PER_S)])
    pltpu.sync_copy(cnt_pk, cnt_sh.at[pl.ds(sid * _NCPAD, _NCPAD)])
    pltpu.sync_copy(start_pk, start_sh.at[pl.ds(sid * _NCPAD, _NCPAD)])
    plsc.subcore_barrier()
    pltpu.sync_copy(lists_sh, lists_all)
    pltpu.sync_copy(cnt_sh, cnt_all)
    pltpu.sync_copy(start_sh, start_all)

    bufs = (buf0, buf1)
    sems = (sem0, sem1)

    def _extract_bucket(k, src_load, sw):
        base_ck = sw * _NCPAD + k
        cnt16 = plsc.load_gather(cnt_all, [jnp.broadcast_to(base_ck, (_L,))])
        st16 = plsc.load_gather(start_all, [jnp.broadcast_to(base_ck, (_L,))])
        lbase = sw * _B_PER_S
        cnt = jnp.max(cnt16)
        st = jnp.max(st16)

        def _ext(jv, _2):
            lane = jv * _L + iota
            ent16 = plsc.load_gather(lists_all, [lbase + st + lane])
            lmask = lane < cnt16
            off16 = lax.shift_right_logical(ent16, 16)
            b16 = ent16 & 0xFFFF
            v = src_load(off16, lmask)
            plsc.store_scatter(vals_v, [b16], v, mask=lmask)
            return _2

        lax.fori_loop(0, (cnt + _L - 1) // _L, _ext, 0)

    for j in range(_CPS):
        comp = core * (_NS * _CPS) + sid * _CPS + j
        g = comp // 8
        s = comp % 8
        col = t3_hbm.at[g, s]
        pltpu.async_copy(col.at[pl.ds(0, _CH)], buf0, sem0)

        def _chunk_pair(k2, _, col=col):
            for slot in range(2):
                k = k2 * 2 + slot

                @pl.when(k + 1 < _NMAIN)
                def _prefetch(k=k, slot=slot, col=col):
                    pltpu.async_copy(
                        col.at[pl.ds((k + 1) * _CH, _CH)],
                        bufs[1 - slot], sems[1 - slot],
                    )

                pltpu.make_async_copy(
                    col.at[pl.ds(k * _CH, _CH)], bufs[slot], sems[slot]
                ).wait()
                for sw in range(_NS):
                    _extract_bucket(
                        k,
                        lambda o16, m, slot=slot: plsc.load_gather(
                            bufs[slot], [o16], mask=m),
                        sw,
                    )
            return _

        lax.fori_loop(0, _NMAIN // 2, _chunk_pair, 0)
        comp_v = jnp.broadcast_to(comp, (_L,)).astype(jnp.int32)
        for sw in range(_NS):
            _extract_bucket(
                _NMAIN,
                lambda o16, m: plsc.load_gather(
                    tail_v, [o16, comp_v], mask=m),
                sw,
            )
        pltpu.sync_copy(vals_v, out_hbm.at[g, s])


def kernel(x, embedding_location):
    location_idx = x[:, 0]
    t3 = embedding_location.T.reshape(8, 8, NUM_LOCATION)
    tail = embedding_location[_TAIL0:, :]
    out_t = _lookup_kernel(location_idx, t3, tail)
    return out_t.reshape(EMBEDDING_DIM, BATCH).T


# speedup vs baseline: 3.9200x; 1.1717x over previous
import functools

import jax
import jax.numpy as jnp
from jax import lax
from jax.experimental import pallas as pl
from jax.experimental.pallas import tpu as pltpu
from jax.experimental.pallas import tpu_sc as plsc

BATCH = 16384
EMBEDDING_DIM = 64
NUM_LOCATION = 1000000

_info = plsc.get_sparse_core_info()
_NC, _NS = _info.num_cores, _info.num_subcores
_L = 16
_B_PER_S = BATCH // _NS
_CH = 27776
_NMAIN = 36
_TAIL0 = _NMAIN * _CH
_NTAIL = NUM_LOCATION - _TAIL0
_NCHUNK = _NMAIN + 1
_NCPAD = 48
_SUPW = 5
_NSUP = 8
_CH1 = _SUPW * _CH
_CPS = EMBEDDING_DIM // _NC // _NS


@functools.partial(
    pl.kernel,
    mesh=plsc.VectorSubcoreMesh(core_axis_name="c", subcore_axis_name="s"),
    out_type=jax.ShapeDtypeStruct((8, 8, BATCH), jnp.float32),
    scratch_types=[
        pltpu.VMEM((_B_PER_S,), jnp.int32),
        pltpu.VMEM((_B_PER_S + _L,), jnp.int32),
        pltpu.VMEM((_B_PER_S + _L,), jnp.int32),
        pltpu.VMEM((_NSUP * _L,), jnp.int32),
        pltpu.VMEM((_NSUP * _L,), jnp.int32),
        pltpu.VMEM((_NCHUNK * _L,), jnp.int32),
        pltpu.VMEM((_NCHUNK * _L,), jnp.int32),
        pltpu.VMEM((_NCPAD,), jnp.int32),
        pltpu.VMEM((_NS * _B_PER_S + _L,), jnp.int32),
        pltpu.VMEM((_NS * _NCPAD,), jnp.int32),
        pltpu.VMEM((_NCPAD,), jnp.int32),
        pltpu.VMEM((_NCPAD,), jnp.int32),
        pltpu.VMEM((_NCPAD,), jnp.int32),
        pltpu.VMEM((_L,), jnp.int32),
        pltpu.VMEM((_L,), jnp.int32),
        pltpu.VMEM((_CH,), jnp.float32),
        pltpu.VMEM((_CH,), jnp.float32),
        pltpu.VMEM((BATCH,), jnp.float32),
        pltpu.VMEM((_NTAIL, EMBEDDING_DIM), jnp.float32),
        pltpu.VMEM_SHARED((_NS * _B_PER_S + _L,), jnp.int32),
        pltpu.VMEM_SHARED((_NS * _NCPAD,), jnp.int32),
        pltpu.SemaphoreType.DMA,
        pltpu.SemaphoreType.DMA,
    ],
    compiler_params=pltpu.CompilerParams(
        use_tc_tiling_on_sc=True, needs_layout_passes=False
    ),
)
def _lookup_kernel(
    idx_hbm, t3_hbm, tail_hbm, out_hbm,
    idx_own, mybuf, supbuf, supcnt, supstart, mycnt, mystart, cnt_pk,
    glist_all, cnt_all, gtot_pk, gst_pk, mybase_pk, pos_stage, ent_stage,
    buf0, buf1, vals_v, tail_v, glist_sh, cnt_sh, sem0, sem1,
):
    core = lax.axis_index("c")
    sid = lax.axis_index("s")
    iota = lax.broadcasted_iota(jnp.int32, (_L,), 0)

    pltpu.sync_copy(idx_hbm.at[pl.ds(sid * _B_PER_S, _B_PER_S)], idx_own)
    pltpu.sync_copy(tail_hbm, tail_v)
    b_base = sid * _B_PER_S

    def _super(k1, pos):
        lo = jnp.broadcast_to(k1 * _CH1, (_L,)).astype(jnp.int32)
        hi = jnp.minimum(lo + _CH1, NUM_LOCATION)
        pos_start = pos

        def _step(i, p):
            idx16 = idx_own[pl.ds(i * _L, _L)]
            m = (idx16 >= lo) & (idx16 < hi)
            ent16 = lax.shift_left(idx16 - lo, 14) | (b_base + i * _L + iota)
            ranks = plsc.cumsum(m.astype(jnp.int32))
            plsc.store_scatter(supbuf, [p + ranks - 1], ent16, mask=m)
            return p + jnp.max(ranks)

        pos = lax.fori_loop(0, _B_PER_S // _L, _step, pos)
        supcnt[pl.ds(k1 * _L, _L)] = jnp.broadcast_to(pos - pos_start, (_L,))
        supstart[pl.ds(k1 * _L, _L)] = jnp.broadcast_to(pos_start, (_L,))
        return pos

    lax.fori_loop(0, _NSUP, _super, jnp.int32(0))

    def _split_super(k1, pos):
        scnt16 = supcnt[pl.ds(k1 * _L, _L)]
        sst16 = supstart[pl.ds(k1 * _L, _L)]
        scnt = jnp.max(scnt16)
        sst = jnp.max(sst16)
        nvec = (scnt + _L - 1) // _L

        def _sub(j2, pos):
            k = k1 * _SUPW + j2
            lo = jnp.broadcast_to(j2 * _CH, (_L,)).astype(jnp.int32)
            hi = jnp.minimum(lo + _CH, NUM_LOCATION - k1 * _CH1)
            pos_start = pos

            def _step2(i, p):
                lane = i * _L + iota
                ent = plsc.load_gather(supbuf, [sst + lane])
                loc = lax.shift_right_logical(ent, 14)
                m = (lane < scnt16) & (loc >= lo) & (loc < hi)
                ent2 = lax.shift_left(loc - lo, 16) | (ent & 0x3FFF)
                ranks = plsc.cumsum(m.astype(jnp.int32))
                plsc.store_scatter(mybuf, [p + ranks - 1], ent2, mask=m)
                return p + jnp.max(ranks)

            pos = lax.fori_loop(0, nvec, _step2, pos)
            mycnt[pl.ds(k * _L, _L)] = jnp.broadcast_to(pos - pos_start, (_L,))
            mystart[pl.ds(k * _L, _L)] = jnp.broadcast_to(pos_start, (_L,))
            return pos

        return lax.fori_loop(
            0, jnp.minimum(_SUPW, _NCHUNK - k1 * _SUPW), _sub, pos)

    lax.fori_loop(0, _NSUP, _split_super, jnp.int32(0))
    for t in range(_NCPAD // _L):
        sel = jnp.minimum((t * _L + iota) * _L, (_NCHUNK - 1) * _L)
        cnt_pk[pl.ds(t * _L, _L)] = plsc.load_gather(mycnt, [sel])

    pltpu.sync_copy(cnt_pk, cnt_sh.at[pl.ds(sid * _NCPAD, _NCPAD)])
    plsc.subcore_barrier()
    pltpu.sync_copy(cnt_sh, cnt_all)

    def _offsets(k, gs):
        cnt16k = plsc.load_gather(cnt_all, [iota * _NCPAD + k])
        total = jnp.sum(cnt16k)
        myoff = jnp.sum(jnp.where(iota < sid, cnt16k, 0))
        k16 = jnp.broadcast_to(k, (_L,))
        plsc.store_scatter(gtot_pk, [k16], jnp.broadcast_to(total, (_L,)))
        plsc.store_scatter(gst_pk, [k16], jnp.broadcast_to(gs, (_L,)))
        plsc.store_scatter(
            mybase_pk, [k16], jnp.broadcast_to(gs + myoff, (_L,)))
        return gs + total

    lax.fori_loop(0, _NCHUNK, _offsets, jnp.int32(0))

    def _scatter_mine(k, _):
        k16 = jnp.broadcast_to(k, (_L,))
        base16 = plsc.load_gather(mybase_pk, [k16])
        cntm16 = plsc.load_gather(mycnt, [k16 * _L])
        stm = jnp.max(plsc.load_gather(mystart, [k16 * _L]))
        cntm = jnp.max(cntm16)

        def _pub(jv, _2):
            lane = jv * _L + iota
            ent16 = plsc.load_gather(mybuf, [stm + lane])
            lmask = lane < cntm16
            pos16 = jnp.where(lmask, base16 + lane, _NS * _B_PER_S)
            pos_stage[pl.ds(0, _L)] = pos16
            ent_stage[pl.ds(0, _L)] = ent16
            pltpu.sync_copy(ent_stage, glist_sh.at[pos_stage])
            return _2

        lax.fori_loop(0, (cntm + _L - 1) // _L, _pub, 0)
        return _

    lax.fori_loop(0, _NCHUNK, _scatter_mine, 0)
    plsc.subcore_barrier()
    pltpu.sync_copy(glist_sh, glist_all)

    bufs = (buf0, buf1)
    sems = (sem0, sem1)

    def _extract_bucket(k, src_load):
        k16 = jnp.broadcast_to(k, (_L,))
        cnt16 = plsc.load_gather(gtot_pk, [k16])
        st16 = plsc.load_gather(gst_pk, [k16])
        cnt = jnp.max(cnt16)
        st = jnp.max(st16)

        def _ext(jv, _2):
            lane = jv * _L + iota
            ent16 = plsc.load_gather(glist_all, [st + lane])
            lmask = lane < cnt16
            off16 = lax.shift_right_logical(ent16, 16)
            b16 = ent16 & 0xFFFF
            v = src_load(off16, lmask)
            plsc.store_scatter(vals_v, [b16], v, mask=lmask)
            return _2

        lax.fori_loop(0, (cnt + _L - 1) // _L, _ext, 0)

    for j in range(_CPS):
        comp = core * (_NS * _CPS) + sid * _CPS + j
        g = comp // 8
        s = comp % 8
        col = t3_hbm.at[g, s]
        pltpu.async_copy(col.at[pl.ds(0, _CH)], buf0, sem0)

        def _chunk_pair(k2, _, col=col):
            for slot in range(2):
                k = k2 * 2 + slot

                @pl.when(k + 1 < _NMAIN)
                def _prefetch(k=k, slot=slot, col=col):
                    pltpu.async_copy(
                        col.at[pl.ds((k + 1) * _CH, _CH)],
                        bufs[1 - slot], sems[1 - slot],
                    )

                pltpu.make_async_copy(
                    col.at[pl.ds(k * _CH, _CH)], bufs[slot], sems[slot]
                ).wait()
                _extract_bucket(
                    k,
                    lambda o16, m, slot=slot: plsc.load_gather(
                        bufs[slot], [o16], mask=m),
                )
            return _

        lax.fori_loop(0, _NMAIN // 2, _chunk_pair, 0)
        comp_v = jnp.broadcast_to(comp, (_L,)).astype(jnp.int32)
        _extract_bucket(
            _NMAIN,
            lambda o16, m: plsc.load_gather(tail_v, [o16, comp_v], mask=m),
        )
        pltpu.sync_copy(vals_v, out_hbm.at[g, s])


def kernel(x, embedding_location):
    location_idx = x[:, 0]
    t3 = embedding_location.T.reshape(8, 8, NUM_LOCATION)
    tail = embedding_location[_TAIL0:, :]
    out_t = _lookup_kernel(location_idx, t3, tail)
    return out_t.reshape(EMBEDDING_DIM, BATCH).T


# speedup vs baseline: 4.3739x; 1.1158x over previous
import functools

import jax
import jax.numpy as jnp
from jax import lax
from jax.experimental import pallas as pl
from jax.experimental.pallas import tpu as pltpu
from jax.experimental.pallas import tpu_sc as plsc

BATCH = 16384
EMBEDDING_DIM = 64
NUM_LOCATION = 1000000

_info = plsc.get_sparse_core_info()
_NC, _NS = _info.num_cores, _info.num_subcores
_L = 16
_B_PER_S = BATCH // _NS
_CH = 27776
_NMAIN = 36
_TAIL0 = _NMAIN * _CH
_NTAIL = NUM_LOCATION - _TAIL0
_NCHUNK = _NMAIN + 1
_NCPAD = 48
_SUPW = 5
_NSUP = 8
_CH1 = _SUPW * _CH
_CPS = EMBEDDING_DIM // _NC // _NS


@functools.partial(
    pl.kernel,
    mesh=plsc.VectorSubcoreMesh(core_axis_name="c", subcore_axis_name="s"),
    out_type=jax.ShapeDtypeStruct((8, 8, BATCH), jnp.float32),
    scratch_types=[
        pltpu.VMEM((_B_PER_S,), jnp.int32),
        pltpu.VMEM((_B_PER_S + _L,), jnp.int32),
        pltpu.VMEM((_B_PER_S + _L,), jnp.int32),
        pltpu.VMEM((_NSUP * _L,), jnp.int32),
        pltpu.VMEM((_NSUP * _L,), jnp.int32),
        pltpu.VMEM((_NCHUNK * _L,), jnp.int32),
        pltpu.VMEM((_NCHUNK * _L,), jnp.int32),
        pltpu.VMEM((_NCPAD,), jnp.int32),
        pltpu.VMEM((_NS * _B_PER_S + _L,), jnp.int32),
        pltpu.VMEM((_NS * _NCPAD,), jnp.int32),
        pltpu.VMEM((_NCPAD,), jnp.int32),
        pltpu.VMEM((_NCPAD,), jnp.int32),
        pltpu.VMEM((_NCPAD,), jnp.int32),
        pltpu.VMEM((_L,), jnp.int32),
        pltpu.VMEM((_L,), jnp.int32),
        pltpu.VMEM((_CH,), jnp.float32),
        pltpu.VMEM((_CH,), jnp.float32),
        pltpu.VMEM((_CH,), jnp.float32),
        pltpu.VMEM((BATCH,), jnp.float32),
        pltpu.VMEM((_NTAIL,), jnp.float32),
        pltpu.VMEM((_NTAIL,), jnp.float32),
        pltpu.VMEM_SHARED((_NS * _B_PER_S + _L,), jnp.int32),
        pltpu.VMEM_SHARED((_NS * _NCPAD,), jnp.int32),
        pltpu.SemaphoreType.DMA,
        pltpu.SemaphoreType.DMA,
        pltpu.SemaphoreType.DMA,
    ],
    compiler_params=pltpu.CompilerParams(
        use_tc_tiling_on_sc=True, needs_layout_passes=False
    ),
)
def _lookup_kernel(
    idx_hbm, t3_hbm, tail_hbm, out_hbm,
    idx_own, mybuf, supbuf, supcnt, supstart, mycnt, mystart, cnt_pk,
    glist_all, cnt_all, gtot_pk, gst_pk, mybase_pk, pos_stage, ent_stage,
    buf0, buf1, buf2, vals_v, tail0_v, tail1_v, glist_sh, cnt_sh, sem0, sem1, sem2,
):
    core = lax.axis_index("c")
    sid = lax.axis_index("s")
    iota = lax.broadcasted_iota(jnp.int32, (_L,), 0)
    bufs = (buf0, buf1, buf2)
    sems = (sem0, sem1, sem2)

    comp0 = core * (_NS * _CPS) + sid * _CPS
    col0 = t3_hbm.at[comp0 // 8, comp0 % 8]
    pltpu.async_copy(col0.at[pl.ds(0, _CH)], buf0, sem0)
    pltpu.async_copy(col0.at[pl.ds(_CH, _CH)], buf1, sem1)

    pltpu.sync_copy(idx_hbm.at[pl.ds(sid * _B_PER_S, _B_PER_S)], idx_own)
    pltpu.sync_copy(tail_hbm.at[comp0], tail0_v)
    pltpu.sync_copy(tail_hbm.at[comp0 + 1], tail1_v)
    b_base = sid * _B_PER_S

    def _super(k1, pos):
        lo = jnp.broadcast_to(k1 * _CH1, (_L,)).astype(jnp.int32)
        hi = jnp.minimum(lo + _CH1, NUM_LOCATION)
        pos_start = pos

        def _step(i, p):
            idx16 = idx_own[pl.ds(i * _L, _L)]
            m = (idx16 >= lo) & (idx16 < hi)
            ent16 = lax.shift_left(idx16 - lo, 14) | (b_base + i * _L + iota)
            ranks = plsc.cumsum(m.astype(jnp.int32))
            plsc.store_scatter(supbuf, [p + ranks - 1], ent16, mask=m)
            return p + jnp.max(ranks)

        pos = lax.fori_loop(0, _B_PER_S // _L, _step, pos)
        supcnt[pl.ds(k1 * _L, _L)] = jnp.broadcast_to(pos - pos_start, (_L,))
        supstart[pl.ds(k1 * _L, _L)] = jnp.broadcast_to(pos_start, (_L,))
        return pos

    lax.fori_loop(0, _NSUP, _super, jnp.int32(0))

    def _split_super(k1, pos):
        scnt16 = supcnt[pl.ds(k1 * _L, _L)]
        sst16 = supstart[pl.ds(k1 * _L, _L)]
        scnt = jnp.max(scnt16)
        sst = jnp.max(sst16)
        nvec = (scnt + _L - 1) // _L

        def _sub(j2, pos):
            k = k1 * _SUPW + j2
            lo = jnp.broadcast_to(j2 * _CH, (_L,)).astype(jnp.int32)
            hi = jnp.minimum(lo + _CH, NUM_LOCATION - k1 * _CH1)
            pos_start = pos

            def _step2(i, p):
                lane = i * _L + iota
                ent = plsc.load_gather(supbuf, [sst + lane])
                loc = lax.shift_right_logical(ent, 14)
                m = (lane < scnt16) & (loc >= lo) & (loc < hi)
                ent2 = lax.shift_left(loc - lo, 16) | (ent & 0x3FFF)
                ranks = plsc.cumsum(m.astype(jnp.int32))
                plsc.store_scatter(mybuf, [p + ranks - 1], ent2, mask=m)
                return p + jnp.max(ranks)

            pos = lax.fori_loop(0, nvec, _step2, pos)
            mycnt[pl.ds(k * _L, _L)] = jnp.broadcast_to(pos - pos_start, (_L,))
            mystart[pl.ds(k * _L, _L)] = jnp.broadcast_to(pos_start, (_L,))
            return pos

        return lax.fori_loop(
            0, jnp.minimum(_SUPW, _NCHUNK - k1 * _SUPW), _sub, pos)

    lax.fori_loop(0, _NSUP, _split_super, jnp.int32(0))
    for t in range(_NCPAD // _L):
        sel = jnp.minimum((t * _L + iota) * _L, (_NCHUNK - 1) * _L)
        cnt_pk[pl.ds(t * _L, _L)] = plsc.load_gather(mycnt, [sel])

    pltpu.sync_copy(cnt_pk, cnt_sh.at[pl.ds(sid * _NCPAD, _NCPAD)])
    plsc.subcore_barrier()
    pltpu.sync_copy(cnt_sh, cnt_all)

    def _offsets(k, gs):
        cnt16k = plsc.load_gather(cnt_all, [iota * _NCPAD + k])
        total = jnp.sum(cnt16k)
        myoff = jnp.sum(jnp.where(iota < sid, cnt16k, 0))
        k16 = jnp.broadcast_to(k, (_L,))
        plsc.store_scatter(gtot_pk, [k16], jnp.broadcast_to(total, (_L,)))
        plsc.store_scatter(gst_pk, [k16], jnp.broadcast_to(gs, (_L,)))
        plsc.store_scatter(
            mybase_pk, [k16], jnp.broadcast_to(gs + myoff, (_L,)))
        return gs + total

    lax.fori_loop(0, _NCHUNK, _offsets, jnp.int32(0))

    def _scatter_mine(k, _):
        k16 = jnp.broadcast_to(k, (_L,))
        base16 = plsc.load_gather(mybase_pk, [k16])
        cntm16 = plsc.load_gather(mycnt, [k16 * _L])
        stm = jnp.max(plsc.load_gather(mystart, [k16 * _L]))
        cntm = jnp.max(cntm16)

        def _pub(jv, _2):
            lane = jv * _L + iota
            ent16 = plsc.load_gather(mybuf, [stm + lane])
            lmask = lane < cntm16
            pos16 = jnp.where(lmask, base16 + lane, _NS * _B_PER_S)
            pos_stage[pl.ds(0, _L)] = pos16
            ent_stage[pl.ds(0, _L)] = ent16
            pltpu.sync_copy(ent_stage, glist_sh.at[pos_stage])
            return _2

        lax.fori_loop(0, (cntm + _L - 1) // _L, _pub, 0)
        return _

    lax.fori_loop(0, _NCHUNK, _scatter_mine, 0)
    plsc.subcore_barrier()
    pltpu.sync_copy(glist_sh, glist_all)

    def _extract_bucket(k, src_load):
        k16 = jnp.broadcast_to(k, (_L,))
        cnt16 = plsc.load_gather(gtot_pk, [k16])
        st16 = plsc.load_gather(gst_pk, [k16])
        cnt = jnp.max(cnt16)
        st = jnp.max(st16)

        def _ext(jv, _2):
            lane = jv * _L + iota
            ent16 = plsc.load_gather(glist_all, [st + lane])
            lmask = lane < cnt16
            off16 = lax.shift_right_logical(ent16, 16)
            b16 = ent16 & 0xFFFF
            v = src_load(off16, lmask)
            plsc.store_scatter(vals_v, [b16], v, mask=lmask)
            return _2

        lax.fori_loop(0, (cnt + _L - 1) // _L, _ext, 0)

    for j in range(_CPS):
        comp = core * (_NS * _CPS) + sid * _CPS + j
        g = comp // 8
        s = comp % 8
        col = t3_hbm.at[g, s]
        if j > 0:
            pltpu.async_copy(col.at[pl.ds(0, _CH)], buf0, sem0)
            pltpu.async_copy(col.at[pl.ds(_CH, _CH)], buf1, sem1)

        def _chunk_trip(k3, _, col=col):
            for slot in range(3):
                k = k3 * 3 + slot

                @pl.when(k + 2 < _NMAIN)
                def _prefetch(k=k, slot=slot, col=col):
                    pltpu.async_copy(
                        col.at[pl.ds((k + 2) * _CH, _CH)],
                        bufs[(slot + 2) % 3], sems[(slot + 2) % 3],
                    )

                pltpu.make_async_copy(
                    col.at[pl.ds(k * _CH, _CH)], bufs[slot], sems[slot]
                ).wait()
                _extract_bucket(
                    k,
                    lambda o16, m, slot=slot: plsc.load_gather(
                        bufs[slot], [o16], mask=m),
                )
            return _

        lax.fori_loop(0, _NMAIN // 3, _chunk_trip, 0)
        tail_j = (tail0_v, tail1_v)[j]
        _extract_bucket(
            _NMAIN,
            lambda o16, m, tail_j=tail_j: plsc.load_gather(
                tail_j, [o16], mask=m),
        )
        pltpu.sync_copy(vals_v, out_hbm.at[g, s])


def kernel(x, embedding_location):
    location_idx = x[:, 0]
    t3 = embedding_location.T.reshape(8, 8, NUM_LOCATION)
    tail = embedding_location[_TAIL0:, :].T
    out_t = _lookup_kernel(location_idx, t3, tail)
    return out_t.reshape(EMBEDDING_DIM, BATCH).T
